# baseline (speedup 1.0000x reference)
import dataclasses
import functools

import jax
import jax.numpy as jnp
from jax import lax
from jax.experimental import pallas as pl
from jax.experimental.pallas import tpu as pltpu
from jax.experimental.pallas import tpu_sc as plsc

N = 10000
D = 128
E = 320000
NC = 2
NS = 16
NW = NC * NS
L = 16
CH = 64
CPS = 160
DEPTH = 4
GA = 2
E_PAD = NW * CPS * CH
RPS = 624
BM = 1000

_GDN = lax.GatherDimensionNumbers(
    offset_dims=(), collapsed_slice_dims=(0,), start_index_map=(0,))


def _mm_body(x_ref, w_ref, o_ref):
    o_ref[...] = jnp.dot(x_ref[...], w_ref[...],
                         preferred_element_type=jnp.float32)


def _matmul(x, W):
    return pl.pallas_call(
        _mm_body,
        grid=(N // BM,),
        in_specs=[pl.BlockSpec((BM, D), lambda i: (i, 0)),
                  pl.BlockSpec((D, D), lambda i: (0, 0))],
        out_specs=pl.BlockSpec((BM, D), lambda i: (i, 0)),
        out_shape=jax.ShapeDtypeStruct((N, D), jnp.float32),
    )(x, W)


def _fused_mm_body(p_ref, b_ref, w_ref, o_ref):
    h = jnp.maximum(p_ref[0] + p_ref[1] + b_ref[0][None, :], 0.0)
    o_ref[...] = jnp.dot(h, w_ref[...], preferred_element_type=jnp.float32)


def _relu_bias_matmul(p, b, W):
    return pl.pallas_call(
        _fused_mm_body,
        grid=(N // BM,),
        in_specs=[pl.BlockSpec((NC, BM, D), lambda i: (0, i, 0)),
                  pl.BlockSpec((8, D), lambda i: (0, 0)),
                  pl.BlockSpec((D, D), lambda i: (0, 0))],
        out_specs=pl.BlockSpec((BM, D), lambda i: (i, 0)),
        out_shape=jax.ShapeDtypeStruct((N, D), jnp.float32),
    )(p, b, W)


def _bias_add_body(p_ref, b_ref, o_ref):
    o_ref[...] = p_ref[0] + p_ref[1] + b_ref[0][None, :]


def _bias_add(p, b):
    return pl.pallas_call(
        _bias_add_body,
        grid=(N // BM,),
        in_specs=[pl.BlockSpec((NC, BM, D), lambda i: (0, i, 0)),
                  pl.BlockSpec((8, D), lambda i: (0, 0))],
        out_specs=pl.BlockSpec((BM, D), lambda i: (i, 0)),
        out_shape=jax.ShapeDtypeStruct((N, D), jnp.float32),
    )(p, b)


_MESH = plsc.VectorSubcoreMesh(core_axis_name="c", subcore_axis_name="s")

_CP = pltpu.CompilerParams()
if "needs_layout_passes" in pltpu.CompilerParams.__dataclass_fields__:
    _CP = dataclasses.replace(_CP, needs_layout_passes=False)

_SCRATCH = (
    [pltpu.VMEM((2, CH), jnp.int32) for _ in range(DEPTH)]
    + [pltpu.VMEM((1, CH), jnp.int32) for _ in range(DEPTH)]
    + [pltpu.VMEM((CH, D), jnp.float32) for _ in range(DEPTH)]
    + [pltpu.SemaphoreType.DMA for _ in range(4 * DEPTH)]
    + [pltpu.VMEM_SHARED((N, D), jnp.float32)]
)


@functools.partial(
    pl.kernel,
    out_type=jax.ShapeDtypeStruct((NC, N, D), jnp.float32),
    mesh=_MESH,
    compiler_params=_CP,
    scratch_types=_SCRATCH,
)
def _spmm_kernel(dense_hbm, cw_hbm, row_hbm, out_hbm, *scr):
    cid = lax.axis_index("c")
    sid = lax.axis_index("s")
    wid = sid * NC + cid

    cwbuf = scr[:DEPTH]
    ribuf = scr[DEPTH:2 * DEPTH]
    rbuf = scr[2 * DEPTH:3 * DEPTH]
    sems = scr[3 * DEPTH:7 * DEPTH]
    csem = sems[:DEPTH]
    rsem = sems[DEPTH:2 * DEPTH]
    gsem = sems[2 * DEPTH:3 * DEPTH]
    ssem = sems[3 * DEPTH:]
    acc = scr[7 * DEPTH]

    start = wid * CPS

    z = rbuf[0]

    @pl.loop(0, CH)
    def _(r):
        for t in range(D // L):
            z[r, pl.ds(t * L, L)] = jnp.zeros((L,), jnp.float32)

    zbase = sid * RPS
    for k in range(RPS // CH):
        pltpu.sync_copy(z, acc.at[pl.ds(zbase + k * CH, CH)])
    pltpu.sync_copy(z.at[pl.ds(0, RPS % CH)],
                    acc.at[pl.ds(zbase + (RPS // CH) * CH, RPS % CH)])

    @pl.when(sid == NS - 1)
    def _():
        pltpu.sync_copy(z.at[pl.ds(0, 16)], acc.at[pl.ds(N - 16, 16)])

    plsc.subcore_barrier()

    def start_cw(c, b):
        pltpu.async_copy(cw_hbm.at[start + c], cwbuf[b], csem[b])

    def wait_cw(b):
        pltpu.make_async_copy(cw_hbm.at[0], cwbuf[b], csem[b]).wait()

    def start_row(c, b):
        pltpu.async_copy(row_hbm.at[start + c], ribuf[b], rsem[b])

    def wait_row(b):
        pltpu.make_async_copy(row_hbm.at[0], ribuf[b], rsem[b]).wait()

    def start_gather(b):
        pltpu.async_copy(dense_hbm.at[cwbuf[b].at[0]], rbuf[b], gsem[b])

    def wait_gather(b):
        pltpu.make_async_copy(dense_hbm.at[cwbuf[b].at[0]], rbuf[b],
                              gsem[b]).wait()

    def start_scatter(b):
        pltpu.async_copy(rbuf[b], acc.at[ribuf[b].at[0]], ssem[b], add=True)

    def wait_scatter(b):
        pltpu.make_async_copy(rbuf[b], acc.at[ribuf[b].at[0]],
                              ssem[b]).wait()

    def scale(b):
        rv, cw = rbuf[b], cwbuf[b]

        @pl.loop(0, CH, step=L)
        def _(g):
            wg = plsc.bitcast(cw[1, pl.ds(g, L)], jnp.float32)
            for e in range(L):
                idxs = jnp.full((L, 1), e, jnp.int32)
                wv = lax.gather(wg, idxs, _GDN, slice_sizes=(1,),
                                mode=lax.GatherScatterMode.PROMISE_IN_BOUNDS)
                for t in range(D // L):
                    rv[g + e, pl.ds(t * L, L)] = (
                        rv[g + e, pl.ds(t * L, L)] * wv)

    for k in range(DEPTH):
        start_cw(k, k)
    for k in range(GA):
        wait_cw(k)
        start_row(k, k)
        start_gather(k)

    @pl.loop(0, CPS, step=DEPTH)
    def _(c):
        for u in range(DEPTH):
            cc = c + u
            b = u
            gb = (u + GA) % DEPTH

            wait_gather(b)
            scale(b)

            @pl.when(cc + DEPTH < CPS)
            def _():
                start_cw(cc + DEPTH, b)

            wait_row(b)
            start_scatter(b)

            @pl.when(cc >= DEPTH - GA)
            def _():
                wait_scatter(gb)

            @pl.when(cc + GA < CPS)
            def _():
                wait_cw(gb)
                start_row(cc + GA, gb)
                start_gather(gb)

    for k in range(DEPTH - GA):
        wait_scatter((CPS - (DEPTH - GA) + k) % DEPTH)
    plsc.subcore_barrier()

    for k in range(RPS // CH):
        pltpu.sync_copy(acc.at[pl.ds(zbase + k * CH, CH)],
                        out_hbm.at[cid].at[pl.ds(zbase + k * CH, CH)])
    pltpu.sync_copy(acc.at[pl.ds(zbase + (RPS // CH) * CH, RPS % CH)],
                    out_hbm.at[cid].at[pl.ds(zbase + (RPS // CH) * CH,
                                             RPS % CH)])

    @pl.when(sid == NS - 1)
    def _():
        pltpu.sync_copy(acc.at[pl.ds(N - 16, 16)],
                        out_hbm.at[cid].at[pl.ds(N - 16, 16)])


def _spmm_partials(dense, cwp, rowp):
    return _spmm_kernel(dense, cwp, rowp)


def kernel(x, edge_index, edge_weight, W1, b1, W2, b2):
    row = edge_index[0].astype(jnp.int32)
    col = edge_index[1].astype(jnp.int32)
    w = edge_weight.astype(jnp.float32)

    pad = E_PAD - E
    zi = jnp.zeros((pad,), jnp.int32)
    colp = jnp.concatenate([col, zi]).reshape(NW * CPS, 1, CH)
    wbits = lax.bitcast_convert_type(
        jnp.concatenate([w, jnp.zeros((pad,), jnp.float32)]), jnp.int32
    ).reshape(NW * CPS, 1, CH)
    cwp = jnp.concatenate([colp, wbits], axis=1)
    rowp = jnp.concatenate([row, zi]).reshape(NW * CPS, 1, CH)

    b1e = jnp.broadcast_to(b1[None, :], (8, D))
    b2e = jnp.broadcast_to(b2[None, :], (8, D))

    support = _matmul(x, W1)
    p1 = _spmm_partials(support, cwp, rowp)
    support2 = _relu_bias_matmul(p1, b1e, W2)
    p2 = _spmm_partials(support2, cwp, rowp)
    return _bias_add(p2, b2e)


# speedup vs baseline: 1.1186x; 1.1186x over previous
import dataclasses
import functools

import jax
import jax.numpy as jnp
from jax import lax
from jax.experimental import pallas as pl
from jax.experimental.pallas import tpu as pltpu
from jax.experimental.pallas import tpu_sc as plsc

N = 10000
D = 128
E = 320000
NC = 2
NS = 16
NW = NC * NS
L = 16
CH = 64
TCH = 5120
CPS0 = 232
CPS1 = 88
BASE1 = NS * CPS0
DEPTH = 4
GA = 2
E_PAD = TCH * CH
RPS = 624
BM = 1000

_GDN = lax.GatherDimensionNumbers(
    offset_dims=(), collapsed_slice_dims=(0,), start_index_map=(0,))


def _mm_body(x_ref, w_ref, o_ref):
    o_ref[...] = jnp.dot(x_ref[...], w_ref[...],
                         preferred_element_type=jnp.float32)


def _matmul(x, W):
    return pl.pallas_call(
        _mm_body,
        grid=(N // BM,),
        in_specs=[pl.BlockSpec((BM, D), lambda i: (i, 0)),
                  pl.BlockSpec((D, D), lambda i: (0, 0))],
        out_specs=pl.BlockSpec((BM, D), lambda i: (i, 0)),
        out_shape=jax.ShapeDtypeStruct((N, D), jnp.float32),
    )(x, W)


def _fused_mm_body(p_ref, b_ref, w_ref, o_ref):
    h = jnp.maximum(p_ref[0] + p_ref[1] + b_ref[0][None, :], 0.0)
    o_ref[...] = jnp.dot(h, w_ref[...], preferred_element_type=jnp.float32)


def _relu_bias_matmul(p, b, W):
    return pl.pallas_call(
        _fused_mm_body,
        grid=(N // BM,),
        in_specs=[pl.BlockSpec((NC, BM, D), lambda i: (0, i, 0)),
                  pl.BlockSpec((8, D), lambda i: (0, 0)),
                  pl.BlockSpec((D, D), lambda i: (0, 0))],
        out_specs=pl.BlockSpec((BM, D), lambda i: (i, 0)),
        out_shape=jax.ShapeDtypeStruct((N, D), jnp.float32),
    )(p, b, W)


def _bias_add_body(p_ref, b_ref, o_ref):
    o_ref[...] = p_ref[0] + p_ref[1] + b_ref[0][None, :]


def _bias_add(p, b):
    return pl.pallas_call(
        _bias_add_body,
        grid=(N // BM,),
        in_specs=[pl.BlockSpec((NC, BM, D), lambda i: (0, i, 0)),
                  pl.BlockSpec((8, D), lambda i: (0, 0))],
        out_specs=pl.BlockSpec((BM, D), lambda i: (i, 0)),
        out_shape=jax.ShapeDtypeStruct((N, D), jnp.float32),
    )(p, b)


_MESH = plsc.VectorSubcoreMesh(core_axis_name="c", subcore_axis_name="s")

_CP = pltpu.CompilerParams()
if "needs_layout_passes" in pltpu.CompilerParams.__dataclass_fields__:
    _CP = dataclasses.replace(_CP, needs_layout_passes=False)

_SCRATCH = (
    [pltpu.VMEM((2, CH), jnp.int32) for _ in range(DEPTH)]
    + [pltpu.VMEM((1, CH), jnp.int32) for _ in range(DEPTH)]
    + [pltpu.VMEM((CH, D), jnp.float32) for _ in range(DEPTH)]
    + [pltpu.SemaphoreType.DMA for _ in range(4 * DEPTH)]
    + [pltpu.VMEM_SHARED((N, D), jnp.float32)]
)


@functools.partial(
    pl.kernel,
    out_type=jax.ShapeDtypeStruct((NC, N, D), jnp.float32),
    mesh=_MESH,
    compiler_params=_CP,
    scratch_types=_SCRATCH,
)
def _spmm_kernel(dense_hbm, cw_hbm, row_hbm, out_hbm, *scr):
    cid = lax.axis_index("c")
    sid = lax.axis_index("s")

    cwbuf = scr[:DEPTH]
    ribuf = scr[DEPTH:2 * DEPTH]
    rbuf = scr[2 * DEPTH:3 * DEPTH]
    sems = scr[3 * DEPTH:7 * DEPTH]
    csem = sems[:DEPTH]
    rsem = sems[DEPTH:2 * DEPTH]
    gsem = sems[2 * DEPTH:3 * DEPTH]
    ssem = sems[3 * DEPTH:]
    acc = scr[7 * DEPTH]

    z = rbuf[0]

    @pl.loop(0, CH)
    def _(r):
        for t in range(D // L):
            z[r, pl.ds(t * L, L)] = jnp.zeros((L,), jnp.float32)

    zbase = sid * RPS
    for k in range(RPS // CH):
        pltpu.sync_copy(z, acc.at[pl.ds(zbase + k * CH, CH)])
    pltpu.sync_copy(z.at[pl.ds(0, RPS % CH)],
                    acc.at[pl.ds(zbase + (RPS // CH) * CH, RPS % CH)])

    @pl.when(sid == NS - 1)
    def _():
        pltpu.sync_copy(z.at[pl.ds(0, 16)], acc.at[pl.ds(N - 16, 16)])

    plsc.subcore_barrier()

    def make_pipeline(start, cps):
        def start_cw(c, b):
            pltpu.async_copy(cw_hbm.at[start + c], cwbuf[b], csem[b])

        def wait_cw(b):
            pltpu.make_async_copy(cw_hbm.at[0], cwbuf[b], csem[b]).wait()

        def start_row(c, b):
            pltpu.async_copy(row_hbm.at[start + c], ribuf[b], rsem[b])

        def wait_row(b):
            pltpu.make_async_copy(row_hbm.at[0], ribuf[b], rsem[b]).wait()

        def start_gather(b):
            pltpu.async_copy(dense_hbm.at[cwbuf[b].at[0]], rbuf[b], gsem[b])

        def wait_gather(b):
            pltpu.make_async_copy(dense_hbm.at[cwbuf[b].at[0]], rbuf[b],
                                  gsem[b]).wait()

        def start_scatter(b):
            pltpu.async_copy(rbuf[b], acc.at[ribuf[b].at[0]], ssem[b],
                             add=True)

        def wait_scatter(b):
            pltpu.make_async_copy(rbuf[b], acc.at[ribuf[b].at[0]],
                                  ssem[b]).wait()

        def scale(b):
            rv, cw = rbuf[b], cwbuf[b]

            @pl.loop(0, CH, step=L)
            def _(g):
                wg = plsc.bitcast(cw[1, pl.ds(g, L)], jnp.float32)
                for e in range(L):
                    idxs = jnp.full((L, 1), e, jnp.int32)
                    wv = lax.gather(
                        wg, idxs, _GDN, slice_sizes=(1,),
                        mode=lax.GatherScatterMode.PROMISE_IN_BOUNDS)
                    for t in range(D // L):
                        rv[g + e, pl.ds(t * L, L)] = (
                            rv[g + e, pl.ds(t * L, L)] * wv)

        for k in range(DEPTH):
            start_cw(k, k)
        for k in range(GA):
            wait_cw(k)
            start_row(k, k)
            start_gather(k)

        @pl.loop(0, cps, step=DEPTH)
        def _(c):
            for u in range(DEPTH):
                cc = c + u
                b = u
                gb = (u + GA) % DEPTH

                wait_gather(b)
                scale(b)

                @pl.when(cc + DEPTH < cps)
                def _():
                    start_cw(cc + DEPTH, b)

                wait_row(b)
                start_scatter(b)

                @pl.when(cc >= DEPTH - GA)
                def _():
                    wait_scatter(gb)

                @pl.when(cc + GA < cps)
                def _():
                    wait_cw(gb)
                    start_row(cc + GA, gb)
                    start_gather(gb)

        for k in range(DEPTH - GA):
            wait_scatter((cps - (DEPTH - GA) + k) % DEPTH)

    @pl.when(cid == 0)
    def _():
        make_pipeline(sid * CPS0, CPS0)

    @pl.when(cid == 1)
    def _():
        make_pipeline(BASE1 + sid * CPS1, CPS1)

    plsc.subcore_barrier()

    for k in range(RPS // CH):
        pltpu.sync_copy(acc.at[pl.ds(zbase + k * CH, CH)],
                        out_hbm.at[cid].at[pl.ds(zbase + k * CH, CH)])
    pltpu.sync_copy(acc.at[pl.ds(zbase + (RPS // CH) * CH, RPS % CH)],
                    out_hbm.at[cid].at[pl.ds(zbase + (RPS // CH) * CH,
                                             RPS % CH)])

    @pl.when(sid == NS - 1)
    def _():
        pltpu.sync_copy(acc.at[pl.ds(N - 16, 16)],
                        out_hbm.at[cid].at[pl.ds(N - 16, 16)])


def _spmm_partials(dense, cwp, rowp):
    return _spmm_kernel(dense, cwp, rowp)


def kernel(x, edge_index, edge_weight, W1, b1, W2, b2):
    row = edge_index[0].astype(jnp.int32)
    col = edge_index[1].astype(jnp.int32)
    w = edge_weight.astype(jnp.float32)

    pad = E_PAD - E
    zi = jnp.zeros((pad,), jnp.int32)
    colp = jnp.concatenate([col, zi]).reshape(TCH, 1, CH)
    wbits = lax.bitcast_convert_type(
        jnp.concatenate([w, jnp.zeros((pad,), jnp.float32)]), jnp.int32
    ).reshape(TCH, 1, CH)
    cwp = jnp.concatenate([colp, wbits], axis=1)
    rowp = jnp.concatenate([row, zi]).reshape(TCH, 1, CH)

    b1e = jnp.broadcast_to(b1[None, :], (8, D))
    b2e = jnp.broadcast_to(b2[None, :], (8, D))

    support = _matmul(x, W1)
    p1 = _spmm_partials(support, cwp, rowp)
    support2 = _relu_bias_matmul(p1, b1e, W2)
    p2 = _spmm_partials(support2, cwp, rowp)
    return _bias_add(p2, b2e)
